# baseline (speedup 1.0000x reference)
import functools

import jax
import jax.numpy as jnp
import numpy as np
from jax import lax
from jax.experimental import pallas as pl
from jax.experimental.pallas import tpu as pltpu, tpu_sc as plsc

_N_BINS = 15
_NH = 2 * _N_BINS
_NC, _NS, _L = 2, 16, 16
_NW = _NC * _NS
_N = 16777216
_C = 16384
_G = 22
_PER_W = _G * _C
_N_SC = _NW * _PER_W
_VPC = _C // _L
_TCOLS = 65
_SLOTS = 2 * _NH

_ROWS = _N // 1024
_ROW0 = _N_SC // 1024
_BR = 256
_NG = (_ROWS - _ROW0) // _BR
_BINS_LO = np.linspace(0.0, 1.0, _N_BINS + 1, dtype=np.float32)[:-1]


def _ece_body(logits_hbm, labels_hbm, out_hbm,
              l0, l1, b0, b1, tab_cnt, tab_conf, obuf, sem0, sem1):
    wid = lax.axis_index("s") * _NC + lax.axis_index("c")
    base = wid * _PER_W

    zeros16 = jnp.zeros((_L,), jnp.float32)
    ones16 = jnp.ones((_L,), jnp.float32)
    laneoff = lax.iota(jnp.int32, _L) * _TCOLS

    for l in range(_L):
        for q in range(4):
            tab_cnt[pl.ds(l * _TCOLS + 16 * q, 16)] = zeros16
            tab_conf[pl.ds(l * _TCOLS + 16 * q, 16)] = zeros16

    lbufs = (l0, l1)
    bbufs = (b0, b1)
    sems = (sem0, sem1)

    def start(g, par):
        off = base + g * _C
        pltpu.async_copy(logits_hbm.at[pl.ds(off, _C)], lbufs[par], sems[par])
        pltpu.async_copy(labels_hbm.at[pl.ds(off, _C)], bbufs[par], sems[par])

    def wait(g, par):
        off = base + g * _C
        pltpu.make_async_copy(
            logits_hbm.at[pl.ds(off, _C)], lbufs[par], sems[par]).wait()
        pltpu.make_async_copy(
            labels_hbm.at[pl.ds(off, _C)], bbufs[par], sems[par]).wait()

    KV = 16

    idx_pad = laneoff + 62

    def process(par):
        lb = lbufs[par]
        bb = bbufs[par]

        init = (tuple(idx_pad for _ in range(KV)),
                tuple(zeros16 for _ in range(KV)))

        @pl.loop(0, _VPC, step=KV, init_carry=init)
        def scan(j, carry):
            prev_idxs, prev_ps = carry
            off = j * _L
            ps = []
            idxs = []
            for k in range(KV):
                x = lb[pl.ds(off + k * _L, _L)]
                li = bb[pl.ds(off + k * _L, _L)]
                p = 1.0 / (1.0 + jnp.exp(x * jnp.float32(-1.0)))
                hb = (p * jnp.float32(_NH)).astype(jnp.int32)
                ps.append(p)
                idxs.append(hb + hb + li + laneoff)
            for k in range(KV):
                plsc.addupdate_scatter(tab_cnt, [prev_idxs[k]], ones16)
                plsc.addupdate_scatter(tab_conf, [prev_idxs[k]], prev_ps[k])
            return (tuple(idxs), tuple(ps))

        last_idxs, last_ps = scan
        for k in range(KV):
            plsc.addupdate_scatter(tab_cnt, [last_idxs[k]], ones16)
            plsc.addupdate_scatter(tab_conf, [last_idxs[k]], last_ps[k])

    start(0, 0)

    @pl.loop(0, _G, step=2)
    def _(g):
        for par in (0, 1):
            gg = g + par
            nxt = gg + 1

            @pl.when(nxt < _G)
            def _():
                start(nxt, 1 - par)

            wait(gg, par)
            process(par)

    accs = [zeros16] * 8
    for l in range(_L):
        for q in range(4):
            accs[q] = accs[q] + tab_cnt[pl.ds(l * _TCOLS + 16 * q, 16)]
            accs[4 + q] = accs[4 + q] + tab_conf[pl.ds(l * _TCOLS + 16 * q, 16)]
    for q in range(8):
        obuf[pl.ds(16 * q, 16)] = accs[q]
    pltpu.sync_copy(obuf, out_hbm.at[wid])


@functools.partial(
    pl.kernel,
    out_type=jax.ShapeDtypeStruct((_NW, 128), jnp.float32),
    mesh=plsc.VectorSubcoreMesh(core_axis_name="c", subcore_axis_name="s"),
    compiler_params=pltpu.CompilerParams(needs_layout_passes=False),
    scratch_types=[
        pltpu.VMEM((_C,), jnp.float32),
        pltpu.VMEM((_C,), jnp.float32),
        pltpu.VMEM((_C,), jnp.int32),
        pltpu.VMEM((_C,), jnp.int32),
        pltpu.VMEM((_L * _TCOLS,), jnp.float32),
        pltpu.VMEM((_L * _TCOLS,), jnp.float32),
        pltpu.VMEM((128,), jnp.float32),
        pltpu.SemaphoreType.DMA,
        pltpu.SemaphoreType.DMA,
    ],
)
def _ece_partials(logits_hbm, labels_hbm, out_hbm,
                  l0, l1, b0, b1, tab_cnt, tab_conf, obuf, sem0, sem1):
    _ece_body(logits_hbm, labels_hbm, out_hbm,
              l0, l1, b0, b1, tab_cnt, tab_conf, obuf, sem0, sem1)


def _tc_hist_body(xref, lref, oref):
    x = xref[...]
    li = lref[...]
    p = 1.0 / (1.0 + jnp.exp(-x))
    accf = jnp.where((p > 0.5) == (li == 1), 1.0, 0.0)
    sums = []
    for k in range(_N_BINS):
        m = p > _BINS_LO[k]
        sums.append(jnp.sum(jnp.where(m, 1.0, 0.0)))
    sums.append(jnp.float32(0.0))
    for k in range(_N_BINS):
        m = p > _BINS_LO[k]
        sums.append(jnp.sum(jnp.where(m, p, 0.0)))
    sums.append(jnp.float32(0.0))
    for k in range(_N_BINS):
        m = p > _BINS_LO[k]
        sums.append(jnp.sum(jnp.where(m, accf, 0.0)))
    sums.append(jnp.float32(0.0))
    oref[0, 0, :] = jnp.stack(sums)


_tc_partials = pl.pallas_call(
    _tc_hist_body,
    grid=(_NG,),
    in_specs=[
        pl.BlockSpec((_BR, 1024), lambda g: (g + _ROW0 // _BR, 0)),
        pl.BlockSpec((_BR, 1024), lambda g: (g + _ROW0 // _BR, 0)),
    ],
    out_specs=pl.BlockSpec((1, 1, 48), lambda g: (g, 0, 0)),
    out_shape=jax.ShapeDtypeStruct((_NG, 1, 48), jnp.float32),
)


def kernel(logits, labels):
    parts = _ece_partials(logits, labels)
    parts_tc = _tc_partials(logits.reshape(_ROWS, 1024),
                            labels.reshape(_ROWS, 1024))
    s = parts.sum(axis=0)
    cnt_s = s[0:_SLOTS]
    conf_s = s[64:64 + _SLOTS]
    cnt_h = cnt_s[0::2] + cnt_s[1::2]
    conf_h = conf_s[0::2] + conf_s[1::2]
    pred = jnp.arange(_NH) >= (_NH // 2)
    acc_h = jnp.where(pred, cnt_s[1::2], cnt_s[0::2])
    cnt = cnt_h[0::2] + cnt_h[1::2]
    conf_sum = conf_h[0::2] + conf_h[1::2]
    acc_sum = acc_h[0::2] + acc_h[1::2]
    t = parts_tc.sum(axis=(0, 1))
    cnt = cnt + t[0:15] - t[1:16]
    conf_sum = conf_sum + t[16:31] - t[17:32]
    acc_sum = acc_sum + t[32:47] - t[33:48]
    prob = cnt / jnp.float32(_N)
    safe = jnp.maximum(cnt, 1.0)
    has = cnt > 0
    acc_in = jnp.where(has, acc_sum / safe, 0.0)
    conf_in = jnp.where(has, conf_sum / safe, 0.0)
    ece = jnp.sum(jnp.abs(conf_in - acc_in) * prob)
    return ece.reshape((1,))


# speedup vs baseline: 1.7317x; 1.7317x over previous
import functools

import jax
import jax.numpy as jnp
from jax import lax
from jax.experimental import pallas as pl
from jax.experimental.pallas import tpu as pltpu, tpu_sc as plsc

_N_BINS = 15
_NH = 2 * _N_BINS
_NC, _NS, _L = 2, 16, 16
_NW = _NC * _NS
_N = 16777216
_PER_W = _N // _NW
_C = 16384
_G = _PER_W // _C
_VPC = _C // _L
_TCOLS = 65
_SLOTS = 2 * _NH


def _ece_body(logits_hbm, labels_hbm, out_hbm,
              l0, l1, b0, b1, tab_cnt, tab_conf, obuf, sem0, sem1):
    wid = lax.axis_index("s") * _NC + lax.axis_index("c")
    base = wid * _PER_W

    zeros16 = jnp.zeros((_L,), jnp.float32)
    ones16 = jnp.ones((_L,), jnp.float32)
    laneoff = lax.iota(jnp.int32, _L) * _TCOLS

    for l in range(_L):
        for q in range(4):
            tab_cnt[pl.ds(l * _TCOLS + 16 * q, 16)] = zeros16
            tab_conf[pl.ds(l * _TCOLS + 16 * q, 16)] = zeros16

    lbufs = (l0, l1)
    bbufs = (b0, b1)
    sems = (sem0, sem1)

    def start(g, par):
        off = base + g * _C
        pltpu.async_copy(logits_hbm.at[pl.ds(off, _C)], lbufs[par], sems[par])
        pltpu.async_copy(labels_hbm.at[pl.ds(off, _C)], bbufs[par], sems[par])

    def wait(g, par):
        off = base + g * _C
        pltpu.make_async_copy(
            logits_hbm.at[pl.ds(off, _C)], lbufs[par], sems[par]).wait()
        pltpu.make_async_copy(
            labels_hbm.at[pl.ds(off, _C)], bbufs[par], sems[par]).wait()

    KV = 16

    idx_pad = laneoff + 62

    def process(par):
        lb = lbufs[par]
        bb = bbufs[par]

        init = (tuple(idx_pad for _ in range(KV)),
                tuple(zeros16 for _ in range(KV)))

        @pl.loop(0, _VPC, step=KV, init_carry=init)
        def scan(j, carry):
            prev_idxs, prev_ps = carry
            off = j * _L
            ps = []
            idxs = []
            for k in range(KV):
                x = lb[pl.ds(off + k * _L, _L)]
                li = bb[pl.ds(off + k * _L, _L)]
                p = 1.0 / (1.0 + jnp.exp(x * jnp.float32(-1.0)))
                hb = (p * jnp.float32(_NH)).astype(jnp.int32)
                ps.append(p)
                idxs.append(hb + hb + li + laneoff)
            for k in range(KV):
                plsc.addupdate_scatter(tab_cnt, [prev_idxs[k]], ones16)
                plsc.addupdate_scatter(tab_conf, [prev_idxs[k]], prev_ps[k])
            return (tuple(idxs), tuple(ps))

        last_idxs, last_ps = scan
        for k in range(KV):
            plsc.addupdate_scatter(tab_cnt, [last_idxs[k]], ones16)
            plsc.addupdate_scatter(tab_conf, [last_idxs[k]], last_ps[k])

    start(0, 0)

    @pl.loop(0, _G, step=2)
    def _(g):
        for par in (0, 1):
            gg = g + par
            nxt = gg + 1

            @pl.when(nxt < _G)
            def _():
                start(nxt, 1 - par)

            wait(gg, par)
            process(par)

    accs = [zeros16] * 8
    for l in range(_L):
        for q in range(4):
            accs[q] = accs[q] + tab_cnt[pl.ds(l * _TCOLS + 16 * q, 16)]
            accs[4 + q] = accs[4 + q] + tab_conf[pl.ds(l * _TCOLS + 16 * q, 16)]
    for q in range(8):
        obuf[pl.ds(16 * q, 16)] = accs[q]
    pltpu.sync_copy(obuf, out_hbm.at[wid])


@functools.partial(
    pl.kernel,
    out_type=jax.ShapeDtypeStruct((_NW, 128), jnp.float32),
    mesh=plsc.VectorSubcoreMesh(core_axis_name="c", subcore_axis_name="s"),
    compiler_params=pltpu.CompilerParams(needs_layout_passes=False),
    scratch_types=[
        pltpu.VMEM((_C,), jnp.float32),
        pltpu.VMEM((_C,), jnp.float32),
        pltpu.VMEM((_C,), jnp.int32),
        pltpu.VMEM((_C,), jnp.int32),
        pltpu.VMEM((_L * _TCOLS,), jnp.float32),
        pltpu.VMEM((_L * _TCOLS,), jnp.float32),
        pltpu.VMEM((128,), jnp.float32),
        pltpu.SemaphoreType.DMA,
        pltpu.SemaphoreType.DMA,
    ],
)
def _ece_partials(logits_hbm, labels_hbm, out_hbm,
                  l0, l1, b0, b1, tab_cnt, tab_conf, obuf, sem0, sem1):
    _ece_body(logits_hbm, labels_hbm, out_hbm,
              l0, l1, b0, b1, tab_cnt, tab_conf, obuf, sem0, sem1)


def kernel(logits, labels):
    parts = _ece_partials(logits, labels)
    s = parts.sum(axis=0)
    cnt_s = s[0:_SLOTS]
    conf_s = s[64:64 + _SLOTS]
    cnt_h = cnt_s[0::2] + cnt_s[1::2]
    conf_h = conf_s[0::2] + conf_s[1::2]
    pred = jnp.arange(_NH) >= (_NH // 2)
    acc_h = jnp.where(pred, cnt_s[1::2], cnt_s[0::2])
    cnt = cnt_h[0::2] + cnt_h[1::2]
    conf_sum = conf_h[0::2] + conf_h[1::2]
    acc_sum = acc_h[0::2] + acc_h[1::2]
    prob = cnt / jnp.float32(_N)
    safe = jnp.maximum(cnt, 1.0)
    has = cnt > 0
    acc_in = jnp.where(has, acc_sum / safe, 0.0)
    conf_in = jnp.where(has, conf_sum / safe, 0.0)
    ece = jnp.sum(jnp.abs(conf_in - acc_in) * prob)
    return ece.reshape((1,))
